# baseline (speedup 1.0000x reference)
import functools

import jax
import jax.numpy as jnp
from jax import lax
from jax.experimental import pallas as pl
from jax.experimental.pallas import tpu as pltpu
from jax.experimental.pallas import tpu_sc as plsc

BATCH = 16384
FEAT = 512
NC = 2
NS = 16
NW = NC * NS
ROWS_PER_W = BATCH // NW
CH = 16
NCHUNK = ROWS_PER_W // CH
NBUF = 4
LANES = 16
VECS_PER_ROW = FEAT // LANES


def _sc_body(feat_hbm, lab_hbm, cent_hbm, out_hbm,
             idx_v, rows_v, feat_v, out_v, *sems):
    wid = lax.axis_index("s") * NC + lax.axis_index("c")
    base = pl.multiple_of(wid * ROWS_PER_W, ROWS_PER_W)
    sems_g = sems[:NBUF]
    sems_f = sems[NBUF:]

    pltpu.sync_copy(lab_hbm.at[pl.ds(base, ROWS_PER_W)], idx_v)

    def start(c, b):
        r0 = pl.multiple_of(c * CH, CH)
        pltpu.async_copy(cent_hbm.at[idx_v.at[pl.ds(r0, CH)]],
                         rows_v.at[b], sems_g[b])
        pltpu.async_copy(feat_hbm.at[pl.ds(base + r0, CH)],
                         feat_v.at[b], sems_f[b])

    def wait(b):
        pltpu.make_async_copy(cent_hbm.at[pl.ds(0, CH)],
                              rows_v.at[b], sems_g[b]).wait()
        pltpu.make_async_copy(feat_hbm.at[pl.ds(0, CH)],
                              feat_v.at[b], sems_f[b]).wait()

    def compute(b, accs):
        def row_body(r, a):
            acc = list(a)
            for t in range(VECS_PER_ROW):
                f = feat_v[b, r, pl.ds(t * LANES, LANES)]
                cv = rows_v[b, r, pl.ds(t * LANES, LANES)]
                d = f - cv
                acc[t % 4] = acc[t % 4] + d * d
            return tuple(acc)
        return lax.fori_loop(0, CH, row_body, accs)

    for c in range(NBUF - 1):
        start(c, c)
    zero = jnp.zeros((LANES,), jnp.float32)

    def outer(g, accs):
        c0 = g * NBUF
        for b in range(NBUF):
            c = c0 + b
            nxt = c + NBUF - 1

            @pl.when(nxt < NCHUNK)
            def _():
                start(nxt, (b + NBUF - 1) % NBUF)

            wait(b)
            accs = compute(b, accs)
        return accs

    a0, a1, a2, a3 = lax.fori_loop(0, NCHUNK // NBUF, outer,
                                   (zero, zero, zero, zero))
    out_v[...] = (a0 + a1) + (a2 + a3)
    pltpu.sync_copy(out_v, out_hbm.at[wid])


@jax.jit
def _center_loss_partials(features, labels, centers):
    mesh = plsc.VectorSubcoreMesh(core_axis_name="c", subcore_axis_name="s")
    run = pl.kernel(
        _sc_body,
        mesh=mesh,
        out_type=jax.ShapeDtypeStruct((NW, LANES), jnp.float32),
        scratch_types=[
            pltpu.VMEM((ROWS_PER_W,), jnp.int32),
            pltpu.VMEM((NBUF, CH, FEAT), jnp.float32),
            pltpu.VMEM((NBUF, CH, FEAT), jnp.float32),
            pltpu.VMEM((LANES,), jnp.float32),
        ] + [pltpu.SemaphoreType.DMA] * (2 * NBUF),
    )
    return run(features, labels, centers)


def kernel(features, labels, centers):
    partials = _center_loss_partials(
        features, labels.astype(jnp.int32), centers)
    return jnp.sum(partials) / jnp.float32(BATCH * FEAT)
